# baseline (speedup 1.0000x reference)
import functools

import jax
import jax.numpy as jnp
from jax import lax
from jax.experimental import pallas as pl
from jax.experimental.pallas import tpu as pltpu
from jax.experimental.pallas import tpu_sc as plsc

N = 10000
E = 160000
D = 256
H = 4
Ch = 64

NP = 10240
EP = 163840
CHK = 128
NW = 32
EW = EP // NW
NCW = EW // CHK
NT = 16
ET = EP // NT
NCT = ET // CHK
RZ = NP // 16


@functools.cache
def _mesh():
    return plsc.VectorSubcoreMesh(core_axis_name="c", subcore_axis_name="s")


def _sc(out_type, scratch_types):
    def deco(body):
        @functools.cache
        def build():
            return pl.kernel(body, out_type=out_type, mesh=_mesh(),
                             scratch_types=scratch_types)

        def call(*args):
            return build()(*args)

        return call
    return deco


def _fill(ref, rows, cols, value):
    v16 = jnp.full((16,), value, jnp.float32)
    cblk = cols // 16

    def body(t, carry):
        ref[t // cblk, pl.ds((t % cblk) * 16, 16)] = v16
        return carry

    lax.fori_loop(0, rows * cblk, body, 0)


def _zero_acc(acc, zbuf, s):
    def zc(b, carry):
        pltpu.sync_copy(zbuf, acc.at[pl.ds(s * RZ + b * CHK, CHK)])
        return carry

    lax.fori_loop(0, RZ // CHK, zc, 0)


def _shift_idx(idx, off):
    def body(g, carry):
        idx[pl.ds(g * 16, 16)] = idx[pl.ds(g * 16, 16)] + off
        return carry

    lax.fori_loop(0, CHK // 16, body, 0)


@_sc(
    out_type=jax.ShapeDtypeStruct((2, NP, 128), jnp.float32),
    scratch_types=[
        pltpu.VMEM_SHARED((NP, 128), jnp.float32),
        pltpu.VMEM((CHK, 128), jnp.float32),
        pltpu.VMEM((CHK,), jnp.int32),
    ],
)
def _sc_deg(dst_hbm, out_hbm, acc, buf, idx):
    c = lax.axis_index("c")
    s = lax.axis_index("s")
    wid = s * 2 + c
    _fill(buf, CHK, 128, 0.0)
    _zero_acc(acc, buf, s)
    plsc.subcore_barrier()
    _fill(buf, CHK, 128, 1.0)

    def step(t, carry):
        base = wid * EW + t * CHK
        pltpu.sync_copy(dst_hbm.at[pl.ds(base, CHK)], idx)
        pltpu.sync_copy(buf, acc.at[idx], add=True)
        return carry

    lax.fori_loop(0, NCW, step, 0)
    plsc.subcore_barrier()
    pltpu.sync_copy(acc.at[pl.ds(s * RZ, RZ)], out_hbm.at[c, pl.ds(s * RZ, RZ)])


@_sc(
    out_type=[
        jax.ShapeDtypeStruct((EP, D), jnp.float32),
        jax.ShapeDtypeStruct((EP, D), jnp.float32),
        jax.ShapeDtypeStruct((EP, D), jnp.float32),
    ],
    scratch_types=[
        pltpu.VMEM((CHK,), jnp.int32),
        pltpu.VMEM((CHK,), jnp.int32),
        pltpu.VMEM((CHK, D), jnp.float32),
        pltpu.VMEM((CHK, D), jnp.float32),
        pltpu.VMEM((CHK, D), jnp.float32),
        pltpu.SemaphoreType.DMA,
        pltpu.SemaphoreType.DMA,
        pltpu.SemaphoreType.DMA,
    ],
)
def _sc_gather_qkv(dst_hbm, src_hbm, q_hbm, k_hbm, v_hbm, qd_hbm, ks_hbm,
                   vs_hbm, idxd, idxs, qbuf, kbuf, vbuf, sq, sk, sv):
    c = lax.axis_index("c")
    s = lax.axis_index("s")
    wid = s * 2 + c

    def step(t, carry):
        base = wid * EW + t * CHK
        pltpu.sync_copy(dst_hbm.at[pl.ds(base, CHK)], idxd)
        pltpu.sync_copy(src_hbm.at[pl.ds(base, CHK)], idxs)
        cq = pltpu.async_copy(q_hbm.at[idxd], qbuf, sq)
        ck = pltpu.async_copy(k_hbm.at[idxs], kbuf, sk)
        cv = pltpu.async_copy(v_hbm.at[idxs], vbuf, sv)
        cq.wait()
        ck.wait()
        cv.wait()
        pltpu.sync_copy(qbuf, qd_hbm.at[pl.ds(base, CHK)])
        pltpu.sync_copy(kbuf, ks_hbm.at[pl.ds(base, CHK)])
        pltpu.sync_copy(vbuf, vs_hbm.at[pl.ds(base, CHK)])
        return carry

    lax.fori_loop(0, NCW, step, 0)


@_sc(
    out_type=jax.ShapeDtypeStruct((2, NP, 128), jnp.float32),
    scratch_types=[
        pltpu.VMEM_SHARED((NP, 128), jnp.float32),
        pltpu.VMEM((CHK, 128), jnp.float32),
        pltpu.VMEM((CHK,), jnp.int32),
        pltpu.VMEM((CHK,), jnp.int32),
    ],
)
def _sc_local(src_hbm, dst_hbm, xs_hbm, out_hbm, acc, rbuf, idxs, idxd):
    c = lax.axis_index("c")
    s = lax.axis_index("s")
    off = c * NP
    _fill(rbuf, CHK, 128, 0.0)
    _zero_acc(acc, rbuf, s)
    plsc.subcore_barrier()

    def step(t, carry):
        base = s * ET + t * CHK
        pltpu.sync_copy(src_hbm.at[pl.ds(base, CHK)], idxs)
        pltpu.sync_copy(dst_hbm.at[pl.ds(base, CHK)], idxd)
        _shift_idx(idxs, off)
        pltpu.sync_copy(xs_hbm.at[idxs], rbuf)
        pltpu.sync_copy(rbuf, acc.at[idxd], add=True)
        return carry

    lax.fori_loop(0, NCT, step, 0)
    plsc.subcore_barrier()
    pltpu.sync_copy(acc.at[pl.ds(s * RZ, RZ)], out_hbm.at[c, pl.ds(s * RZ, RZ)])


@_sc(
    out_type=jax.ShapeDtypeStruct((2, NP, 128), jnp.float32),
    scratch_types=[
        pltpu.VMEM_SHARED((NP, 128), jnp.float32),
        pltpu.VMEM((CHK, 128), jnp.float32),
        pltpu.VMEM((CHK, 16), jnp.float32),
        pltpu.VMEM((CHK,), jnp.int32),
    ],
)
def _sc_scatter16(dst_hbm, val_hbm, out_hbm, acc, rbuf, vbuf, idx):
    c = lax.axis_index("c")
    s = lax.axis_index("s")
    _fill(rbuf, CHK, 128, 0.0)
    _zero_acc(acc, rbuf, s)
    plsc.subcore_barrier()

    wid = s * 2 + c

    def step(t, carry):
        base = wid * EW + t * CHK
        pltpu.sync_copy(dst_hbm.at[pl.ds(base, CHK)], idx)
        pltpu.sync_copy(val_hbm.at[pl.ds(base, CHK)], vbuf)

        def mv(r, cc):
            rbuf[r, pl.ds(0, 16)] = vbuf[r, :]
            return cc

        lax.fori_loop(0, CHK, mv, 0)
        pltpu.sync_copy(rbuf, acc.at[idx], add=True)
        return carry

    lax.fori_loop(0, NCW, step, 0)
    plsc.subcore_barrier()
    pltpu.sync_copy(acc.at[pl.ds(s * RZ, RZ)], out_hbm.at[c, pl.ds(s * RZ, RZ)])


@_sc(
    out_type=jax.ShapeDtypeStruct((2, NP, 128), jnp.float32),
    scratch_types=[
        pltpu.VMEM_SHARED((NP, 128), jnp.float32),
        pltpu.VMEM((CHK, 128), jnp.float32),
        pltpu.VMEM((CHK,), jnp.int32),
    ],
)
def _sc_agg(dst_hbm, wv_hbm, out_hbm, acc, rbuf, idxd):
    c = lax.axis_index("c")
    s = lax.axis_index("s")
    off = c * EP
    _fill(rbuf, CHK, 128, 0.0)
    _zero_acc(acc, rbuf, s)
    plsc.subcore_barrier()

    def step(t, carry):
        base = s * ET + t * CHK
        pltpu.sync_copy(dst_hbm.at[pl.ds(base, CHK)], idxd)
        pltpu.sync_copy(wv_hbm.at[pl.ds(off + base, CHK)], rbuf)
        pltpu.sync_copy(rbuf, acc.at[idxd], add=True)
        return carry

    lax.fori_loop(0, NCT, step, 0)
    plsc.subcore_barrier()
    pltpu.sync_copy(acc.at[pl.ds(s * RZ, RZ)], out_hbm.at[c, pl.ds(s * RZ, RZ)])


_RB = 512
_EB = 1024


def _mm_body(x_ref, w_ref, b_ref, xw_ref, q_ref, k_ref, v_ref, r_ref):
    y = jnp.dot(x_ref[...], w_ref[...], preferred_element_type=jnp.float32)
    y = y + b_ref[...]
    xw_ref[...] = y[:, 0:128]
    q_ref[...] = y[:, 128:256] * 0.125
    k_ref[...] = y[:, 256:384]
    v_ref[...] = y[:, 384:512]
    r_ref[...] = y[:, 512:640]


def _mm(xp, Wcat, bcat):
    nb = NP // _RB
    return pl.pallas_call(
        _mm_body,
        grid=(nb, 2),
        in_specs=[
            pl.BlockSpec((_RB, D), lambda i, j: (i, 0)),
            pl.BlockSpec((D, 5 * 128), lambda i, j: (0, j)),
            pl.BlockSpec((1, 5 * 128), lambda i, j: (0, j)),
        ],
        out_specs=[
            pl.BlockSpec((_RB, 128), lambda i, j: (j * nb + i, 0)),
            pl.BlockSpec((_RB, 128), lambda i, j: (i, j)),
            pl.BlockSpec((_RB, 128), lambda i, j: (i, j)),
            pl.BlockSpec((_RB, 128), lambda i, j: (i, j)),
            pl.BlockSpec((_RB, 128), lambda i, j: (i, j)),
        ],
        out_shape=[
            jax.ShapeDtypeStruct((2 * NP, 128), jnp.float32),
            jax.ShapeDtypeStruct((NP, D), jnp.float32),
            jax.ShapeDtypeStruct((NP, D), jnp.float32),
            jax.ShapeDtypeStruct((NP, D), jnp.float32),
            jax.ShapeDtypeStruct((NP, D), jnp.float32),
        ],
    )(xp, Wcat, bcat)


def _mid_body(degp_ref, xw_ref, dis_ref, xs_ref):
    dp = degp_ref[...]
    deg = dp[0, :, 0:1] + dp[1, :, 0:1]
    dis = jnp.where(deg > 0, 1.0 / jnp.sqrt(jnp.where(deg > 0, deg, 1.0)), 0.0)
    dis_ref[...] = dis
    xs_ref[...] = xw_ref[...] * dis


def _mid(degp, xw_tall):
    nb = NP // _RB
    return pl.pallas_call(
        _mid_body,
        grid=(nb, 2),
        in_specs=[
            pl.BlockSpec((2, _RB, 128), lambda i, j: (0, i, 0)),
            pl.BlockSpec((_RB, 128), lambda i, j: (j * nb + i, 0)),
        ],
        out_specs=[
            pl.BlockSpec((_RB, 1), lambda i, j: (i, 0)),
            pl.BlockSpec((_RB, 128), lambda i, j: (j * nb + i, 0)),
        ],
        out_shape=[
            jax.ShapeDtypeStruct((NP, 1), jnp.float32),
            jax.ShapeDtypeStruct((2 * NP, 128), jnp.float32),
        ],
    )(degp, xw_tall)


def _logits_body(qd_ref, ks_ref, out_ref):
    p = qd_ref[...] * ks_ref[...]
    cols = [jnp.sum(p[:, h * Ch:(h + 1) * Ch], axis=1, keepdims=True)
            for h in range(H)]
    out_ref[...] = jnp.concatenate(cols, axis=1)


def _logits(qd, ks):
    nb = EP // _EB
    return pl.pallas_call(
        _logits_body,
        grid=(nb,),
        in_specs=[
            pl.BlockSpec((_EB, D), lambda i: (i, 0)),
            pl.BlockSpec((_EB, D), lambda i: (i, 0)),
        ],
        out_specs=pl.BlockSpec((_EB, H), lambda i: (i, 0)),
        out_shape=jax.ShapeDtypeStruct((EP, H), jnp.float32),
    )(qd, ks)


def _gmax_body(l_ref, out_ref):
    i = pl.program_id(0)
    m = jnp.max(l_ref[...]).reshape(1, 1)

    @pl.when(i == 0)
    def _():
        out_ref[...] = m

    @pl.when(i > 0)
    def _():
        out_ref[...] = jnp.maximum(out_ref[...], m)


def _gmax(lg):
    nb = EP // _EB
    return pl.pallas_call(
        _gmax_body,
        grid=(nb,),
        in_specs=[pl.BlockSpec((_EB, H), lambda i: (i, 0))],
        out_specs=pl.BlockSpec((1, 1), lambda i: (0, 0)),
        out_shape=jax.ShapeDtypeStruct((1, 1), jnp.float32),
    )(lg)


def _ex_body(l_ref, m_ref, out_ref):
    ex = jnp.exp(l_ref[...] - m_ref[0, 0])
    z6 = jnp.zeros((ex.shape[0], 6), jnp.float32)
    out_ref[...] = jnp.concatenate([ex[:, 0:2], z6, ex[:, 2:4], z6], axis=1)


def _ex(lg, m):
    nb = EP // _EB
    return pl.pallas_call(
        _ex_body,
        grid=(nb,),
        in_specs=[
            pl.BlockSpec((_EB, H), lambda i: (i, 0)),
            pl.BlockSpec((1, 1), lambda i: (0, 0)),
        ],
        out_specs=pl.BlockSpec((_EB, 16), lambda i: (i, 0)),
        out_shape=jax.ShapeDtypeStruct((EP, 16), jnp.float32),
    )(lg, m)


def _den_body(ssp_ref, out_ref):
    sp = ssp_ref[0][:, :16] + ssp_ref[1][:, :16]
    s = jnp.concatenate(
        [sp[:, 0:2], sp[:, 8:10], jnp.ones((sp.shape[0], 12), jnp.float32)],
        axis=1)
    out_ref[...] = jnp.where(s > 0, s, 1.0)


def _den(ssp):
    nb = NP // _RB
    return pl.pallas_call(
        _den_body,
        grid=(nb,),
        in_specs=[pl.BlockSpec((2, _RB, 128), lambda i: (0, i, 0))],
        out_specs=pl.BlockSpec((_RB, 16), lambda i: (i, 0)),
        out_shape=jax.ShapeDtypeStruct((NP, 16), jnp.float32),
    )(ssp)


def _wv_body(vs_ref, ex_ref, wv_ref):
    j = pl.program_id(1)
    exf = ex_ref[...]
    ex = jnp.where(j == 0, exf[:, 0:2], exf[:, 8:10])
    factor = jnp.concatenate(
        [jnp.broadcast_to(ex[:, h:h + 1], (ex.shape[0], Ch))
         for h in range(2)], axis=1)
    wv_ref[...] = vs_ref[...] * factor


def _wv(vs, ex16):
    nb = EP // _EB
    return pl.pallas_call(
        _wv_body,
        grid=(nb, 2),
        in_specs=[
            pl.BlockSpec((_EB, 128), lambda i, j: (i, j)),
            pl.BlockSpec((_EB, 16), lambda i, j: (i, 0)),
        ],
        out_specs=pl.BlockSpec((_EB, 128), lambda i, j: (j * nb + i, 0)),
        out_shape=jax.ShapeDtypeStruct((2 * EP, 128), jnp.float32),
    )(vs, ex16)


def _ln(h, g, b):
    mu = jnp.mean(h, axis=1, keepdims=True)
    var = jnp.mean((h - mu) ** 2, axis=1, keepdims=True)
    return (h - mu) / jnp.sqrt(var + 1e-5) * g + b


def _final_body(loc_ref, aggp_ref, den_ref, dis_ref, r_ref, bg_ref, wb_ref,
                g1_ref, b1_ref, g2_ref, b2_ref, wrel_ref, wroot_ref, lw_ref,
                gw_ref, out_ref):
    lp = loc_ref[...]
    local = jnp.concatenate([lp[0], lp[1]], axis=1)
    local = local * dis_ref[...] + bg_ref[...]
    ap = aggp_ref[...]
    agg = jnp.concatenate([ap[0], ap[1]], axis=1)
    den = den_ref[...][:, :H]
    dfac = jnp.concatenate(
        [jnp.broadcast_to(den[:, h:h + 1], (den.shape[0], Ch))
         for h in range(H)], axis=1)
    agg = agg / dfac
    rr = r_ref[...]
    wb = wb_ref[...]
    wa = wb[0:D] + wb[2 * D:3 * D]
    wr2 = wb[D:2 * D] - wb[2 * D:3 * D]
    z = (jnp.dot(agg, wa, preferred_element_type=jnp.float32)
         + jnp.dot(rr, wr2, preferred_element_type=jnp.float32))
    beta = jax.nn.sigmoid(z)
    glob = beta * rr + (1.0 - beta) * agg
    h = lw_ref[0, 0] * local + gw_ref[0, 0] * glob
    hln = _ln(h + h, g1_ref[...], b1_ref[...])
    f = jnp.maximum(
        jnp.dot(hln, wrel_ref[...], preferred_element_type=jnp.float32), 0.0)
    f = jnp.dot(f, wroot_ref[...], preferred_element_type=jnp.float32)
    out_ref[...] = _ln(f + hln, g2_ref[...], b2_ref[...])


def _final(loc, aggp, den, dis, rt, b_gcn, Wbeta, ln1_g, ln1_b, ln2_g, ln2_b,
           W_rel, W_root, lw, gw):
    fb = 400
    nb = N // fb
    return pl.pallas_call(
        _final_body,
        grid=(nb,),
        in_specs=[
            pl.BlockSpec((2, fb, 128), lambda i: (0, i, 0)),
            pl.BlockSpec((2, fb, 128), lambda i: (0, i, 0)),
            pl.BlockSpec((fb, 16), lambda i: (i, 0)),
            pl.BlockSpec((fb, 1), lambda i: (i, 0)),
            pl.BlockSpec((fb, D), lambda i: (i, 0)),
            pl.BlockSpec((1, D), lambda i: (0, 0)),
            pl.BlockSpec((3 * D, 1), lambda i: (0, 0)),
            pl.BlockSpec((1, D), lambda i: (0, 0)),
            pl.BlockSpec((1, D), lambda i: (0, 0)),
            pl.BlockSpec((1, D), lambda i: (0, 0)),
            pl.BlockSpec((1, D), lambda i: (0, 0)),
            pl.BlockSpec((D, 2 * D), lambda i: (0, 0)),
            pl.BlockSpec((2 * D, D), lambda i: (0, 0)),
            pl.BlockSpec((1, 1), lambda i: (0, 0)),
            pl.BlockSpec((1, 1), lambda i: (0, 0)),
        ],
        out_specs=pl.BlockSpec((fb, D), lambda i: (i, 0)),
        out_shape=jax.ShapeDtypeStruct((N, D), jnp.float32),
    )(loc, aggp, den, dis, rt, b_gcn.reshape(1, D), Wbeta,
      ln1_g.reshape(1, D), ln1_b.reshape(1, D), ln2_g.reshape(1, D),
      ln2_b.reshape(1, D), W_rel, W_root, lw.reshape(1, 1), gw.reshape(1, 1))


def kernel(x, edge_index, W_gcn, b_gcn, Wq, bq, Wk, bk, Wv, bv, Wskip, bskip,
           Wbeta, ln1_g, ln1_b, ln2_g, ln2_b, W_rel, W_root, lw, gw):
    src = edge_index[0]
    dst = edge_index[1]
    pad = jnp.full((EP - E,), N, jnp.int32)
    srcp = jnp.concatenate([src, pad])
    dstp = jnp.concatenate([dst, pad])
    xp = jnp.pad(x, ((0, NP - N), (0, 0)))
    Wcat = jnp.concatenate([W_gcn, Wq, Wk, Wv, Wskip], axis=1)
    Wcat = Wcat.reshape(D, 5, 2, 128).transpose(0, 2, 1, 3).reshape(D, 10 * 128)
    bcat = jnp.concatenate(
        [jnp.zeros_like(b_gcn), bq, bk, bv, bskip]).reshape(5, 2, 128)
    bcat = bcat.transpose(1, 0, 2).reshape(1, 10 * 128)

    xw_t, q_t, k_t, v_t, rt = _mm(xp, Wcat, bcat)
    degp = _sc_deg(dstp)
    dis, xs_t = _mid(degp, xw_t)
    qd, ks, vs = _sc_gather_qkv(dstp, srcp, q_t, k_t, v_t)
    loc = _sc_local(srcp, dstp, xs_t)
    lg = _logits(qd, ks)
    m = _gmax(lg)
    ex16 = _ex(lg, m)
    ssp = _sc_scatter16(dstp, ex16)
    den = _den(ssp)
    wv = _wv(vs, ex16)
    aggp = _sc_agg(dstp, wv)
    return _final(loc, aggp, den, dis, rt, b_gcn, Wbeta, ln1_g, ln1_b, ln2_g,
                  ln2_b, W_rel, W_root, lw, gw)


# speedup vs baseline: 1.0572x; 1.0572x over previous
import functools

import jax
import jax.numpy as jnp
from jax import lax
from jax.experimental import pallas as pl
from jax.experimental.pallas import tpu as pltpu
from jax.experimental.pallas import tpu_sc as plsc

N = 10000
E = 160000
D = 256
H = 4
Ch = 64

NP = 10240
EP = 163840
CHK = 128
NW = 32
EW = EP // NW
NCW = EW // CHK
NT = 16
ET = EP // NT
NCT = ET // CHK
RZ = NP // 16


@functools.cache
def _mesh():
    return plsc.VectorSubcoreMesh(core_axis_name="c", subcore_axis_name="s")


def _sc(out_type, scratch_types):
    def deco(body):
        @functools.cache
        def build():
            return pl.kernel(body, out_type=out_type, mesh=_mesh(),
                             scratch_types=scratch_types)

        def call(*args):
            return build()(*args)

        return call
    return deco


def _fill(ref, rows, cols, value):
    v16 = jnp.full((16,), value, jnp.float32)
    cblk = cols // 16

    def body(t, carry):
        ref[t // cblk, pl.ds((t % cblk) * 16, 16)] = v16
        return carry

    lax.fori_loop(0, rows * cblk, body, 0)


def _zero_acc(acc, zbuf, s):
    def zc(b, carry):
        pltpu.sync_copy(zbuf, acc.at[pl.ds(s * RZ + b * CHK, CHK)])
        return carry

    lax.fori_loop(0, RZ // CHK, zc, 0)


def _shift_idx(idx, off):
    def body(g, carry):
        idx[pl.ds(g * 16, 16)] = idx[pl.ds(g * 16, 16)] + off
        return carry

    lax.fori_loop(0, CHK // 16, body, 0)


@_sc(
    out_type=jax.ShapeDtypeStruct((2, NP, 128), jnp.float32),
    scratch_types=[
        pltpu.VMEM_SHARED((NP, 128), jnp.float32),
        pltpu.VMEM((CHK, 128), jnp.float32),
        pltpu.VMEM((CHK,), jnp.int32),
    ],
)
def _sc_deg(dst_hbm, out_hbm, acc, buf, idx):
    c = lax.axis_index("c")
    s = lax.axis_index("s")
    wid = s * 2 + c
    _fill(buf, CHK, 128, 0.0)
    _zero_acc(acc, buf, s)
    plsc.subcore_barrier()
    _fill(buf, CHK, 128, 1.0)

    def step(t, carry):
        base = wid * EW + t * CHK
        pltpu.sync_copy(dst_hbm.at[pl.ds(base, CHK)], idx)
        pltpu.sync_copy(buf, acc.at[idx], add=True)
        return carry

    lax.fori_loop(0, NCW, step, 0)
    plsc.subcore_barrier()
    pltpu.sync_copy(acc.at[pl.ds(s * RZ, RZ)], out_hbm.at[c, pl.ds(s * RZ, RZ)])


@_sc(
    out_type=[
        jax.ShapeDtypeStruct((EP, D), jnp.float32),
        jax.ShapeDtypeStruct((EP, D), jnp.float32),
        jax.ShapeDtypeStruct((EP, D), jnp.float32),
    ],
    scratch_types=[
        pltpu.VMEM((CHK,), jnp.int32),
        pltpu.VMEM((CHK,), jnp.int32),
        pltpu.VMEM((CHK, D), jnp.float32),
        pltpu.VMEM((CHK, D), jnp.float32),
        pltpu.VMEM((CHK, D), jnp.float32),
        pltpu.SemaphoreType.DMA,
        pltpu.SemaphoreType.DMA,
        pltpu.SemaphoreType.DMA,
    ],
)
def _sc_gather_qkv(dst_hbm, src_hbm, q_hbm, k_hbm, v_hbm, qd_hbm, ks_hbm,
                   vs_hbm, idxd, idxs, qbuf, kbuf, vbuf, sq, sk, sv):
    c = lax.axis_index("c")
    s = lax.axis_index("s")
    wid = s * 2 + c

    def step(t, carry):
        base = wid * EW + t * CHK
        pltpu.sync_copy(dst_hbm.at[pl.ds(base, CHK)], idxd)
        pltpu.sync_copy(src_hbm.at[pl.ds(base, CHK)], idxs)
        cq = pltpu.async_copy(q_hbm.at[idxd], qbuf, sq)
        ck = pltpu.async_copy(k_hbm.at[idxs], kbuf, sk)
        cv = pltpu.async_copy(v_hbm.at[idxs], vbuf, sv)
        cq.wait()
        ck.wait()
        cv.wait()
        pltpu.sync_copy(qbuf, qd_hbm.at[pl.ds(base, CHK)])
        pltpu.sync_copy(kbuf, ks_hbm.at[pl.ds(base, CHK)])
        pltpu.sync_copy(vbuf, vs_hbm.at[pl.ds(base, CHK)])
        return carry

    lax.fori_loop(0, NCW, step, 0)


@_sc(
    out_type=jax.ShapeDtypeStruct((2, NP, 128), jnp.float32),
    scratch_types=[
        pltpu.VMEM_SHARED((NP, 128), jnp.float32),
        pltpu.VMEM((CHK, 128), jnp.float32),
        pltpu.VMEM((CHK,), jnp.int32),
        pltpu.VMEM((CHK,), jnp.int32),
    ],
)
def _sc_local(src_hbm, dst_hbm, xs_hbm, out_hbm, acc, rbuf, idxs, idxd):
    c = lax.axis_index("c")
    s = lax.axis_index("s")
    off = c * NP
    _fill(rbuf, CHK, 128, 0.0)
    _zero_acc(acc, rbuf, s)
    plsc.subcore_barrier()

    def step(t, carry):
        base = s * ET + t * CHK
        pltpu.sync_copy(src_hbm.at[pl.ds(base, CHK)], idxs)
        pltpu.sync_copy(dst_hbm.at[pl.ds(base, CHK)], idxd)
        _shift_idx(idxs, off)
        pltpu.sync_copy(xs_hbm.at[idxs], rbuf)
        pltpu.sync_copy(rbuf, acc.at[idxd], add=True)
        return carry

    lax.fori_loop(0, NCT, step, 0)
    plsc.subcore_barrier()
    pltpu.sync_copy(acc.at[pl.ds(s * RZ, RZ)], out_hbm.at[c, pl.ds(s * RZ, RZ)])


@_sc(
    out_type=jax.ShapeDtypeStruct((2, NP, 128), jnp.float32),
    scratch_types=[
        pltpu.VMEM_SHARED((NP, 128), jnp.float32),
        pltpu.VMEM((CHK, 128), jnp.float32),
        pltpu.VMEM((CHK, 16), jnp.float32),
        pltpu.VMEM((CHK,), jnp.int32),
    ],
)
def _sc_scatter16(dst_hbm, val_hbm, out_hbm, acc, rbuf, vbuf, idx):
    c = lax.axis_index("c")
    s = lax.axis_index("s")
    _fill(rbuf, CHK, 128, 0.0)
    _zero_acc(acc, rbuf, s)
    plsc.subcore_barrier()

    wid = s * 2 + c

    def step(t, carry):
        base = wid * EW + t * CHK
        pltpu.sync_copy(dst_hbm.at[pl.ds(base, CHK)], idx)
        pltpu.sync_copy(val_hbm.at[pl.ds(base, CHK)], vbuf)

        def mv(r, cc):
            rbuf[r, pl.ds(0, 16)] = vbuf[r, :]
            return cc

        lax.fori_loop(0, CHK, mv, 0)
        pltpu.sync_copy(rbuf, acc.at[idx], add=True)
        return carry

    lax.fori_loop(0, NCW, step, 0)
    plsc.subcore_barrier()
    pltpu.sync_copy(acc.at[pl.ds(s * RZ, RZ)], out_hbm.at[c, pl.ds(s * RZ, RZ)])


@_sc(
    out_type=jax.ShapeDtypeStruct((2, 2, NP, 128), jnp.float32),
    scratch_types=[
        pltpu.VMEM_SHARED((NP, 128), jnp.float32),
        pltpu.VMEM((CHK, 128), jnp.float32),
        pltpu.VMEM((CHK, 128), jnp.float32),
        pltpu.VMEM((CHK,), jnp.int32),
    ],
)
def _sc_agg(dst_hbm, wv0_hbm, wv1_hbm, out_hbm, acc, zbuf, rbuf, idxd):
    c = lax.axis_index("c")
    s = lax.axis_index("s")
    wid = s * 2 + c
    _fill(zbuf, CHK, 128, 0.0)
    for h, wv_hbm in ((0, wv0_hbm), (1, wv1_hbm)):
        _zero_acc(acc, zbuf, s)
        plsc.subcore_barrier()

        def step(t, carry):
            base = wid * EW + t * CHK
            pltpu.sync_copy(dst_hbm.at[pl.ds(base, CHK)], idxd)
            pltpu.sync_copy(wv_hbm.at[pl.ds(base, CHK)], rbuf)
            pltpu.sync_copy(rbuf, acc.at[idxd], add=True)
            return carry

        lax.fori_loop(0, NCW, step, 0)
        plsc.subcore_barrier()
        pltpu.sync_copy(acc.at[pl.ds(s * RZ, RZ)],
                        out_hbm.at[c, h, pl.ds(s * RZ, RZ)])


_RB = 512
_EB = 1024


def _mm_body(x_ref, w_ref, b_ref, xw_ref, q_ref, k_ref, v_ref, r_ref):
    y = jnp.dot(x_ref[...], w_ref[...], preferred_element_type=jnp.float32)
    y = y + b_ref[...]
    xw_ref[...] = y[:, 0:128]
    q_ref[...] = y[:, 128:256] * 0.125
    k_ref[...] = y[:, 256:384]
    v_ref[...] = y[:, 384:512]
    r_ref[...] = y[:, 512:640]


def _mm(xp, Wcat, bcat):
    nb = NP // _RB
    return pl.pallas_call(
        _mm_body,
        grid=(nb, 2),
        in_specs=[
            pl.BlockSpec((_RB, D), lambda i, j: (i, 0)),
            pl.BlockSpec((D, 5 * 128), lambda i, j: (0, j)),
            pl.BlockSpec((1, 5 * 128), lambda i, j: (0, j)),
        ],
        out_specs=[
            pl.BlockSpec((_RB, 128), lambda i, j: (j * nb + i, 0)),
            pl.BlockSpec((_RB, 128), lambda i, j: (i, j)),
            pl.BlockSpec((_RB, 128), lambda i, j: (i, j)),
            pl.BlockSpec((_RB, 128), lambda i, j: (i, j)),
            pl.BlockSpec((_RB, 128), lambda i, j: (i, j)),
        ],
        out_shape=[
            jax.ShapeDtypeStruct((2 * NP, 128), jnp.float32),
            jax.ShapeDtypeStruct((NP, D), jnp.float32),
            jax.ShapeDtypeStruct((NP, D), jnp.float32),
            jax.ShapeDtypeStruct((NP, D), jnp.float32),
            jax.ShapeDtypeStruct((NP, D), jnp.float32),
        ],
    )(xp, Wcat, bcat)


def _mid_body(degp_ref, xw_ref, dis_ref, xs_ref):
    dp = degp_ref[...]
    deg = dp[0, :, 0:1] + dp[1, :, 0:1]
    dis = jnp.where(deg > 0, 1.0 / jnp.sqrt(jnp.where(deg > 0, deg, 1.0)), 0.0)
    dis_ref[...] = dis
    xs_ref[...] = xw_ref[...] * dis


def _mid(degp, xw_tall):
    nb = NP // _RB
    return pl.pallas_call(
        _mid_body,
        grid=(nb, 2),
        in_specs=[
            pl.BlockSpec((2, _RB, 128), lambda i, j: (0, i, 0)),
            pl.BlockSpec((_RB, 128), lambda i, j: (j * nb + i, 0)),
        ],
        out_specs=[
            pl.BlockSpec((_RB, 1), lambda i, j: (i, 0)),
            pl.BlockSpec((_RB, 128), lambda i, j: (j * nb + i, 0)),
        ],
        out_shape=[
            jax.ShapeDtypeStruct((NP, 1), jnp.float32),
            jax.ShapeDtypeStruct((2 * NP, 128), jnp.float32),
        ],
    )(degp, xw_tall)


def _logits_body(qd_ref, ks_ref, out_ref):
    p = qd_ref[...] * ks_ref[...]
    cols = [jnp.sum(p[:, h * Ch:(h + 1) * Ch], axis=1, keepdims=True)
            for h in range(H)]
    out_ref[...] = jnp.concatenate(cols, axis=1)


def _logits(qd, ks):
    nb = EP // _EB
    return pl.pallas_call(
        _logits_body,
        grid=(nb,),
        in_specs=[
            pl.BlockSpec((_EB, D), lambda i: (i, 0)),
            pl.BlockSpec((_EB, D), lambda i: (i, 0)),
        ],
        out_specs=pl.BlockSpec((_EB, H), lambda i: (i, 0)),
        out_shape=jax.ShapeDtypeStruct((EP, H), jnp.float32),
    )(qd, ks)


def _gmax_body(l_ref, out_ref):
    i = pl.program_id(0)
    m = jnp.max(l_ref[...]).reshape(1, 1)

    @pl.when(i == 0)
    def _():
        out_ref[...] = m

    @pl.when(i > 0)
    def _():
        out_ref[...] = jnp.maximum(out_ref[...], m)


def _gmax(lg):
    nb = EP // _EB
    return pl.pallas_call(
        _gmax_body,
        grid=(nb,),
        in_specs=[pl.BlockSpec((_EB, H), lambda i: (i, 0))],
        out_specs=pl.BlockSpec((1, 1), lambda i: (0, 0)),
        out_shape=jax.ShapeDtypeStruct((1, 1), jnp.float32),
    )(lg)


def _ex_body(l_ref, m_ref, out_ref):
    ex = jnp.exp(l_ref[...] - m_ref[0, 0])
    out_ref[...] = jnp.concatenate(
        [ex, jnp.zeros((ex.shape[0], 16 - H), jnp.float32)], axis=1)


def _ex(lg, m):
    nb = EP // _EB
    return pl.pallas_call(
        _ex_body,
        grid=(nb,),
        in_specs=[
            pl.BlockSpec((_EB, H), lambda i: (i, 0)),
            pl.BlockSpec((1, 1), lambda i: (0, 0)),
        ],
        out_specs=pl.BlockSpec((_EB, 16), lambda i: (i, 0)),
        out_shape=jax.ShapeDtypeStruct((EP, 16), jnp.float32),
    )(lg, m)


def _den_body(ssp_ref, out_ref):
    s = ssp_ref[0][:, :16] + ssp_ref[1][:, :16]
    out_ref[...] = jnp.where(s > 0, s, 1.0)


def _den(ssp):
    nb = NP // _RB
    return pl.pallas_call(
        _den_body,
        grid=(nb,),
        in_specs=[pl.BlockSpec((2, _RB, 128), lambda i: (0, i, 0))],
        out_specs=pl.BlockSpec((_RB, 16), lambda i: (i, 0)),
        out_shape=jax.ShapeDtypeStruct((NP, 16), jnp.float32),
    )(ssp)


def _wv_body(vs_ref, ex_ref, wv0_ref, wv1_ref):
    ex = ex_ref[...][:, :H]
    factor = jnp.concatenate(
        [jnp.broadcast_to(ex[:, h:h + 1], (ex.shape[0], Ch))
         for h in range(H)], axis=1)
    wv = vs_ref[...] * factor
    wv0_ref[...] = wv[:, :128]
    wv1_ref[...] = wv[:, 128:]


def _wv(vs, ex16):
    nb = EP // _EB
    return pl.pallas_call(
        _wv_body,
        grid=(nb,),
        in_specs=[
            pl.BlockSpec((_EB, D), lambda i: (i, 0)),
            pl.BlockSpec((_EB, 16), lambda i: (i, 0)),
        ],
        out_specs=[
            pl.BlockSpec((_EB, 128), lambda i: (i, 0)),
            pl.BlockSpec((_EB, 128), lambda i: (i, 0)),
        ],
        out_shape=[
            jax.ShapeDtypeStruct((EP, 128), jnp.float32),
            jax.ShapeDtypeStruct((EP, 128), jnp.float32),
        ],
    )(vs, ex16)


def _ln(h, g, b):
    mu = jnp.mean(h, axis=1, keepdims=True)
    var = jnp.mean((h - mu) ** 2, axis=1, keepdims=True)
    return (h - mu) / jnp.sqrt(var + 1e-5) * g + b


def _final_body(loc_ref, aggp_ref, den_ref, dis_ref, r_ref, bg_ref, wb_ref,
                g1_ref, b1_ref, g2_ref, b2_ref, wrel_ref, wroot_ref, lw_ref,
                gw_ref, out_ref):
    lp = loc_ref[...]
    local = jnp.concatenate([lp[0], lp[1]], axis=1)
    local = local * dis_ref[...] + bg_ref[...]
    ap = aggp_ref[...]
    agg = jnp.concatenate([ap[0, 0] + ap[1, 0], ap[0, 1] + ap[1, 1]], axis=1)
    den = den_ref[...][:, :H]
    dfac = jnp.concatenate(
        [jnp.broadcast_to(den[:, h:h + 1], (den.shape[0], Ch))
         for h in range(H)], axis=1)
    agg = agg / dfac
    rr = r_ref[...]
    wb = wb_ref[...]
    wa = wb[0:D] + wb[2 * D:3 * D]
    wr2 = wb[D:2 * D] - wb[2 * D:3 * D]
    z = (jnp.dot(agg, wa, preferred_element_type=jnp.float32)
         + jnp.dot(rr, wr2, preferred_element_type=jnp.float32))
    beta = jax.nn.sigmoid(z)
    glob = beta * rr + (1.0 - beta) * agg
    h = lw_ref[0, 0] * local + gw_ref[0, 0] * glob
    hln = _ln(h + h, g1_ref[...], b1_ref[...])
    f = jnp.maximum(
        jnp.dot(hln, wrel_ref[...], preferred_element_type=jnp.float32), 0.0)
    f = jnp.dot(f, wroot_ref[...], preferred_element_type=jnp.float32)
    out_ref[...] = _ln(f + hln, g2_ref[...], b2_ref[...])


def _final(loc, aggp, den, dis, rt, b_gcn, Wbeta, ln1_g, ln1_b, ln2_g, ln2_b,
           W_rel, W_root, lw, gw):
    fb = 400
    nb = N // fb
    return pl.pallas_call(
        _final_body,
        grid=(nb,),
        in_specs=[
            pl.BlockSpec((2, fb, 128), lambda i: (0, i, 0)),
            pl.BlockSpec((2, 2, fb, 128), lambda i: (0, 0, i, 0)),
            pl.BlockSpec((fb, 16), lambda i: (i, 0)),
            pl.BlockSpec((fb, 1), lambda i: (i, 0)),
            pl.BlockSpec((fb, D), lambda i: (i, 0)),
            pl.BlockSpec((1, D), lambda i: (0, 0)),
            pl.BlockSpec((3 * D, 1), lambda i: (0, 0)),
            pl.BlockSpec((1, D), lambda i: (0, 0)),
            pl.BlockSpec((1, D), lambda i: (0, 0)),
            pl.BlockSpec((1, D), lambda i: (0, 0)),
            pl.BlockSpec((1, D), lambda i: (0, 0)),
            pl.BlockSpec((D, 2 * D), lambda i: (0, 0)),
            pl.BlockSpec((2 * D, D), lambda i: (0, 0)),
            pl.BlockSpec((1, 1), lambda i: (0, 0)),
            pl.BlockSpec((1, 1), lambda i: (0, 0)),
        ],
        out_specs=pl.BlockSpec((fb, D), lambda i: (i, 0)),
        out_shape=jax.ShapeDtypeStruct((N, D), jnp.float32),
    )(loc, aggp, den, dis, rt, b_gcn.reshape(1, D), Wbeta,
      ln1_g.reshape(1, D), ln1_b.reshape(1, D), ln2_g.reshape(1, D),
      ln2_b.reshape(1, D), W_rel, W_root, lw.reshape(1, 1), gw.reshape(1, 1))


def kernel(x, edge_index, W_gcn, b_gcn, Wq, bq, Wk, bk, Wv, bv, Wskip, bskip,
           Wbeta, ln1_g, ln1_b, ln2_g, ln2_b, W_rel, W_root, lw, gw):
    src = edge_index[0]
    dst = edge_index[1]
    pad = jnp.full((EP - E,), N, jnp.int32)
    srcp = jnp.concatenate([src, pad])
    dstp = jnp.concatenate([dst, pad])
    xp = jnp.pad(x, ((0, NP - N), (0, 0)))
    Wcat = jnp.concatenate([W_gcn, Wq, Wk, Wv, Wskip], axis=1)
    Wcat = Wcat.reshape(D, 5, 2, 128).transpose(0, 2, 1, 3).reshape(D, 10 * 128)
    bcat = jnp.concatenate(
        [jnp.zeros_like(b_gcn), bq, bk, bv, bskip]).reshape(5, 2, 128)
    bcat = bcat.transpose(1, 0, 2).reshape(1, 10 * 128)

    xw_t, q_t, k_t, v_t, rt = _mm(xp, Wcat, bcat)
    degp = _sc_deg(dstp)
    dis, xs_t = _mid(degp, xw_t)
    qd, ks, vs = _sc_gather_qkv(dstp, srcp, q_t, k_t, v_t)
    loc = _sc_local(srcp, dstp, xs_t)
    lg = _logits(qd, ks)
    m = _gmax(lg)
    ex16 = _ex(lg, m)
    ssp = _sc_scatter16(dstp, ex16)
    den = _den(ssp)
    wv0, wv1 = _wv(vs, ex16)
    aggp = _sc_agg(dstp, wv0, wv1)
    return _final(loc, aggp, den, dis, rt, b_gcn, Wbeta, ln1_g, ln1_b, ln2_g,
                  ln2_b, W_rel, W_root, lw, gw)


# speedup vs baseline: 1.0822x; 1.0236x over previous
import functools

import jax
import jax.numpy as jnp
from jax import lax
from jax.experimental import pallas as pl
from jax.experimental.pallas import tpu as pltpu
from jax.experimental.pallas import tpu_sc as plsc

N = 10000
E = 160000
D = 256
H = 4
Ch = 64

NP = 10240
EP = 163840
CHK = 128
NW = 32
EW = EP // NW
NCW = EW // CHK
NT = 16
ET = EP // NT
NCT = ET // CHK
RZ = NP // 16


@functools.cache
def _mesh():
    return plsc.VectorSubcoreMesh(core_axis_name="c", subcore_axis_name="s")


def _sc(out_type, scratch_types):
    def deco(body):
        @functools.cache
        def build():
            return pl.kernel(body, out_type=out_type, mesh=_mesh(),
                             scratch_types=scratch_types)

        def call(*args):
            return build()(*args)

        return call
    return deco


def _fill(ref, rows, cols, value):
    v16 = jnp.full((16,), value, jnp.float32)
    cblk = cols // 16

    def body(t, carry):
        ref[t // cblk, pl.ds((t % cblk) * 16, 16)] = v16
        return carry

    lax.fori_loop(0, rows * cblk, body, 0)


def _zero_acc(acc, zbuf, s):
    def zc(b, carry):
        pltpu.sync_copy(zbuf, acc.at[pl.ds(s * RZ + b * CHK, CHK)])
        return carry

    lax.fori_loop(0, RZ // CHK, zc, 0)


def _shift_idx(idx, off):
    def body(g, carry):
        idx[pl.ds(g * 16, 16)] = idx[pl.ds(g * 16, 16)] + off
        return carry

    lax.fori_loop(0, CHK // 16, body, 0)


@_sc(
    out_type=jax.ShapeDtypeStruct((2, NP, 128), jnp.float32),
    scratch_types=[
        pltpu.VMEM_SHARED((NP, 128), jnp.float32),
        pltpu.VMEM((CHK, 128), jnp.float32),
        pltpu.VMEM((CHK,), jnp.int32),
    ],
)
def _sc_deg(dst_hbm, out_hbm, acc, buf, idx):
    c = lax.axis_index("c")
    s = lax.axis_index("s")
    wid = s * 2 + c
    _fill(buf, CHK, 128, 0.0)
    _zero_acc(acc, buf, s)
    plsc.subcore_barrier()
    _fill(buf, CHK, 128, 1.0)

    def step(t, carry):
        base = wid * EW + t * CHK
        pltpu.sync_copy(dst_hbm.at[pl.ds(base, CHK)], idx)
        pltpu.sync_copy(buf, acc.at[idx], add=True)
        return carry

    lax.fori_loop(0, NCW, step, 0)
    plsc.subcore_barrier()
    pltpu.sync_copy(acc.at[pl.ds(s * RZ, RZ)], out_hbm.at[c, pl.ds(s * RZ, RZ)])


@_sc(
    out_type=[
        jax.ShapeDtypeStruct((EP, D), jnp.float32),
        jax.ShapeDtypeStruct((EP, D), jnp.float32),
        jax.ShapeDtypeStruct((EP, D), jnp.float32),
    ],
    scratch_types=[
        pltpu.VMEM((64,), jnp.int32),
        pltpu.VMEM((64,), jnp.int32),
        pltpu.VMEM((64,), jnp.int32),
        pltpu.VMEM((64,), jnp.int32),
        pltpu.VMEM((64, D), jnp.float32),
        pltpu.VMEM((64, D), jnp.float32),
        pltpu.VMEM((64, D), jnp.float32),
        pltpu.VMEM((64, D), jnp.float32),
        pltpu.VMEM((64, D), jnp.float32),
        pltpu.VMEM((64, D), jnp.float32),
        pltpu.SemaphoreType.DMA,
        pltpu.SemaphoreType.DMA,
        pltpu.SemaphoreType.DMA,
        pltpu.SemaphoreType.DMA,
        pltpu.SemaphoreType.DMA,
        pltpu.SemaphoreType.DMA,
        pltpu.SemaphoreType.DMA,
        pltpu.SemaphoreType.DMA,
        pltpu.SemaphoreType.DMA,
        pltpu.SemaphoreType.DMA,
        pltpu.SemaphoreType.DMA,
        pltpu.SemaphoreType.DMA,
    ],
)
def _sc_gather_qkv(dst_hbm, src_hbm, q_hbm, k_hbm, v_hbm, qd_hbm, ks_hbm,
                   vs_hbm, idxd0, idxd1, idxs0, idxs1, qb0, qb1, kb0, kb1,
                   vb0, vb1, gq0, gq1, gk0, gk1, gv0, gv1, wq0, wq1, wk0,
                   wk1, wv0s, wv1s):
    c = lax.axis_index("c")
    s = lax.axis_index("s")
    wid = s * 2 + c
    ck = 64
    nck = EW // ck
    ebase = wid * EW
    sets = (
        (idxd0, idxs0, qb0, kb0, vb0, (gq0, gk0, gv0), (wq0, wk0, wv0s)),
        (idxd1, idxs1, qb1, kb1, vb1, (gq1, gk1, gv1), (wq1, wk1, wv1s)),
    )

    def issue_gathers(t, bset):
        idxd, idxs, qb, kb, vb, gs, _ = bset
        base = ebase + t * ck
        pltpu.sync_copy(dst_hbm.at[pl.ds(base, ck)], idxd)
        pltpu.sync_copy(src_hbm.at[pl.ds(base, ck)], idxs)
        pltpu.async_copy(q_hbm.at[idxd], qb, gs[0])
        pltpu.async_copy(k_hbm.at[idxs], kb, gs[1])
        pltpu.async_copy(v_hbm.at[idxs], vb, gs[2])

    def wait_gathers(bset):
        _, _, qb, kb, vb, gs, _ = bset
        pltpu.make_async_copy(q_hbm.at[pl.ds(0, ck)], qb, gs[0]).wait()
        pltpu.make_async_copy(k_hbm.at[pl.ds(0, ck)], kb, gs[1]).wait()
        pltpu.make_async_copy(v_hbm.at[pl.ds(0, ck)], vb, gs[2]).wait()

    def issue_writes(t, bset):
        _, _, qb, kb, vb, _, ws = bset
        base = ebase + t * ck
        pltpu.async_copy(qb, qd_hbm.at[pl.ds(base, ck)], ws[0])
        pltpu.async_copy(kb, ks_hbm.at[pl.ds(base, ck)], ws[1])
        pltpu.async_copy(vb, vs_hbm.at[pl.ds(base, ck)], ws[2])

    def wait_writes(bset):
        _, _, qb, kb, vb, _, ws = bset
        pltpu.make_async_copy(qb, qd_hbm.at[pl.ds(0, ck)], ws[0]).wait()
        pltpu.make_async_copy(kb, ks_hbm.at[pl.ds(0, ck)], ws[1]).wait()
        pltpu.make_async_copy(vb, vs_hbm.at[pl.ds(0, ck)], ws[2]).wait()

    issue_gathers(0, sets[0])

    def step(i, carry):
        for b in (0, 1):
            t = 2 * i + b
            cur = sets[b]
            nxt = sets[1 - b]
            wait_gathers(cur)

            @pl.when(t >= 1)
            def _():
                wait_writes(nxt)

            @pl.when(t <= nck - 2)
            def _():
                issue_gathers(t + 1, nxt)

            issue_writes(t, cur)
        return carry

    lax.fori_loop(0, nck // 2, step, 0)
    wait_writes(sets[1])


@_sc(
    out_type=jax.ShapeDtypeStruct((2, NP, 128), jnp.float32),
    scratch_types=[
        pltpu.VMEM_SHARED((NP, 128), jnp.float32),
        pltpu.VMEM((CHK, 128), jnp.float32),
        pltpu.VMEM((CHK,), jnp.int32),
        pltpu.VMEM((CHK,), jnp.int32),
    ],
)
def _sc_local(src_hbm, dst_hbm, xs_hbm, out_hbm, acc, rbuf, idxs, idxd):
    c = lax.axis_index("c")
    s = lax.axis_index("s")
    off = c * NP
    _fill(rbuf, CHK, 128, 0.0)
    _zero_acc(acc, rbuf, s)
    plsc.subcore_barrier()

    def step(t, carry):
        base = s * ET + t * CHK
        pltpu.sync_copy(src_hbm.at[pl.ds(base, CHK)], idxs)
        pltpu.sync_copy(dst_hbm.at[pl.ds(base, CHK)], idxd)
        _shift_idx(idxs, off)
        pltpu.sync_copy(xs_hbm.at[idxs], rbuf)
        pltpu.sync_copy(rbuf, acc.at[idxd], add=True)
        return carry

    lax.fori_loop(0, NCT, step, 0)
    plsc.subcore_barrier()
    pltpu.sync_copy(acc.at[pl.ds(s * RZ, RZ)], out_hbm.at[c, pl.ds(s * RZ, RZ)])


@_sc(
    out_type=jax.ShapeDtypeStruct((2, NP, 128), jnp.float32),
    scratch_types=[
        pltpu.VMEM_SHARED((NP, 128), jnp.float32),
        pltpu.VMEM((CHK, 128), jnp.float32),
        pltpu.VMEM((CHK, 16), jnp.float32),
        pltpu.VMEM((CHK,), jnp.int32),
    ],
)
def _sc_scatter16(dst_hbm, val_hbm, out_hbm, acc, rbuf, vbuf, idx):
    c = lax.axis_index("c")
    s = lax.axis_index("s")
    _fill(rbuf, CHK, 128, 0.0)
    _zero_acc(acc, rbuf, s)
    plsc.subcore_barrier()

    wid = s * 2 + c

    def step(t, carry):
        base = wid * EW + t * CHK
        pltpu.sync_copy(dst_hbm.at[pl.ds(base, CHK)], idx)
        pltpu.sync_copy(val_hbm.at[pl.ds(base, CHK)], vbuf)

        def mv(r, cc):
            rbuf[r, pl.ds(0, 16)] = vbuf[r, :]
            return cc

        lax.fori_loop(0, CHK, mv, 0)
        pltpu.sync_copy(rbuf, acc.at[idx], add=True)
        return carry

    lax.fori_loop(0, NCW, step, 0)
    plsc.subcore_barrier()
    pltpu.sync_copy(acc.at[pl.ds(s * RZ, RZ)], out_hbm.at[c, pl.ds(s * RZ, RZ)])


@_sc(
    out_type=jax.ShapeDtypeStruct((2, 2, NP, 128), jnp.float32),
    scratch_types=[
        pltpu.VMEM_SHARED((NP, 128), jnp.float32),
        pltpu.VMEM((CHK, 128), jnp.float32),
        pltpu.VMEM((CHK, 128), jnp.float32),
        pltpu.VMEM((CHK,), jnp.int32),
    ],
)
def _sc_agg(dst_hbm, wv0_hbm, wv1_hbm, out_hbm, acc, zbuf, rbuf, idxd):
    c = lax.axis_index("c")
    s = lax.axis_index("s")
    wid = s * 2 + c
    _fill(zbuf, CHK, 128, 0.0)
    for h, wv_hbm in ((0, wv0_hbm), (1, wv1_hbm)):
        _zero_acc(acc, zbuf, s)
        plsc.subcore_barrier()

        def step(t, carry):
            base = wid * EW + t * CHK
            pltpu.sync_copy(dst_hbm.at[pl.ds(base, CHK)], idxd)
            pltpu.sync_copy(wv_hbm.at[pl.ds(base, CHK)], rbuf)
            pltpu.sync_copy(rbuf, acc.at[idxd], add=True)
            return carry

        lax.fori_loop(0, NCW, step, 0)
        plsc.subcore_barrier()
        pltpu.sync_copy(acc.at[pl.ds(s * RZ, RZ)],
                        out_hbm.at[c, h, pl.ds(s * RZ, RZ)])


_RB = 512
_EB = 1024


def _mm_body(x_ref, w_ref, b_ref, xw_ref, q_ref, k_ref, v_ref, r_ref):
    y = jnp.dot(x_ref[...], w_ref[...], preferred_element_type=jnp.float32)
    y = y + b_ref[...]
    xw_ref[...] = y[:, 0:128]
    q_ref[...] = y[:, 128:256] * 0.125
    k_ref[...] = y[:, 256:384]
    v_ref[...] = y[:, 384:512]
    r_ref[...] = y[:, 512:640]


def _mm(xp, Wcat, bcat):
    nb = NP // _RB
    return pl.pallas_call(
        _mm_body,
        grid=(nb, 2),
        in_specs=[
            pl.BlockSpec((_RB, D), lambda i, j: (i, 0)),
            pl.BlockSpec((D, 5 * 128), lambda i, j: (0, j)),
            pl.BlockSpec((1, 5 * 128), lambda i, j: (0, j)),
        ],
        out_specs=[
            pl.BlockSpec((_RB, 128), lambda i, j: (j * nb + i, 0)),
            pl.BlockSpec((_RB, 128), lambda i, j: (i, j)),
            pl.BlockSpec((_RB, 128), lambda i, j: (i, j)),
            pl.BlockSpec((_RB, 128), lambda i, j: (i, j)),
            pl.BlockSpec((_RB, 128), lambda i, j: (i, j)),
        ],
        out_shape=[
            jax.ShapeDtypeStruct((2 * NP, 128), jnp.float32),
            jax.ShapeDtypeStruct((NP, D), jnp.float32),
            jax.ShapeDtypeStruct((NP, D), jnp.float32),
            jax.ShapeDtypeStruct((NP, D), jnp.float32),
            jax.ShapeDtypeStruct((NP, D), jnp.float32),
        ],
    )(xp, Wcat, bcat)


def _mid_body(degp_ref, xw_ref, dis_ref, xs_ref):
    dp = degp_ref[...]
    deg = dp[0, :, 0:1] + dp[1, :, 0:1]
    dis = jnp.where(deg > 0, 1.0 / jnp.sqrt(jnp.where(deg > 0, deg, 1.0)), 0.0)
    dis_ref[...] = dis
    xs_ref[...] = xw_ref[...] * dis


def _mid(degp, xw_tall):
    nb = NP // _RB
    return pl.pallas_call(
        _mid_body,
        grid=(nb, 2),
        in_specs=[
            pl.BlockSpec((2, _RB, 128), lambda i, j: (0, i, 0)),
            pl.BlockSpec((_RB, 128), lambda i, j: (j * nb + i, 0)),
        ],
        out_specs=[
            pl.BlockSpec((_RB, 1), lambda i, j: (i, 0)),
            pl.BlockSpec((_RB, 128), lambda i, j: (j * nb + i, 0)),
        ],
        out_shape=[
            jax.ShapeDtypeStruct((NP, 1), jnp.float32),
            jax.ShapeDtypeStruct((2 * NP, 128), jnp.float32),
        ],
    )(degp, xw_tall)


def _logits_body(qd_ref, ks_ref, out_ref):
    p = qd_ref[...] * ks_ref[...]
    cols = [jnp.sum(p[:, h * Ch:(h + 1) * Ch], axis=1, keepdims=True)
            for h in range(H)]
    out_ref[...] = jnp.concatenate(cols, axis=1)


def _logits(qd, ks):
    nb = EP // _EB
    return pl.pallas_call(
        _logits_body,
        grid=(nb,),
        in_specs=[
            pl.BlockSpec((_EB, D), lambda i: (i, 0)),
            pl.BlockSpec((_EB, D), lambda i: (i, 0)),
        ],
        out_specs=pl.BlockSpec((_EB, H), lambda i: (i, 0)),
        out_shape=jax.ShapeDtypeStruct((EP, H), jnp.float32),
    )(qd, ks)


def _gmax_body(l_ref, out_ref):
    i = pl.program_id(0)
    m = jnp.max(l_ref[...]).reshape(1, 1)

    @pl.when(i == 0)
    def _():
        out_ref[...] = m

    @pl.when(i > 0)
    def _():
        out_ref[...] = jnp.maximum(out_ref[...], m)


def _gmax(lg):
    nb = EP // _EB
    return pl.pallas_call(
        _gmax_body,
        grid=(nb,),
        in_specs=[pl.BlockSpec((_EB, H), lambda i: (i, 0))],
        out_specs=pl.BlockSpec((1, 1), lambda i: (0, 0)),
        out_shape=jax.ShapeDtypeStruct((1, 1), jnp.float32),
    )(lg)


def _ex_body(l_ref, m_ref, out_ref):
    ex = jnp.exp(l_ref[...] - m_ref[0, 0])
    out_ref[...] = jnp.concatenate(
        [ex, jnp.zeros((ex.shape[0], 16 - H), jnp.float32)], axis=1)


def _ex(lg, m):
    nb = EP // _EB
    return pl.pallas_call(
        _ex_body,
        grid=(nb,),
        in_specs=[
            pl.BlockSpec((_EB, H), lambda i: (i, 0)),
            pl.BlockSpec((1, 1), lambda i: (0, 0)),
        ],
        out_specs=pl.BlockSpec((_EB, 16), lambda i: (i, 0)),
        out_shape=jax.ShapeDtypeStruct((EP, 16), jnp.float32),
    )(lg, m)


def _den_body(ssp_ref, out_ref):
    s = ssp_ref[0][:, :16] + ssp_ref[1][:, :16]
    out_ref[...] = jnp.where(s > 0, s, 1.0)


def _den(ssp):
    nb = NP // _RB
    return pl.pallas_call(
        _den_body,
        grid=(nb,),
        in_specs=[pl.BlockSpec((2, _RB, 128), lambda i: (0, i, 0))],
        out_specs=pl.BlockSpec((_RB, 16), lambda i: (i, 0)),
        out_shape=jax.ShapeDtypeStruct((NP, 16), jnp.float32),
    )(ssp)


def _wv_body(vs_ref, ex_ref, wv0_ref, wv1_ref):
    ex = ex_ref[...][:, :H]
    factor = jnp.concatenate(
        [jnp.broadcast_to(ex[:, h:h + 1], (ex.shape[0], Ch))
         for h in range(H)], axis=1)
    wv = vs_ref[...] * factor
    wv0_ref[...] = wv[:, :128]
    wv1_ref[...] = wv[:, 128:]


def _wv(vs, ex16):
    nb = EP // _EB
    return pl.pallas_call(
        _wv_body,
        grid=(nb,),
        in_specs=[
            pl.BlockSpec((_EB, D), lambda i: (i, 0)),
            pl.BlockSpec((_EB, 16), lambda i: (i, 0)),
        ],
        out_specs=[
            pl.BlockSpec((_EB, 128), lambda i: (i, 0)),
            pl.BlockSpec((_EB, 128), lambda i: (i, 0)),
        ],
        out_shape=[
            jax.ShapeDtypeStruct((EP, 128), jnp.float32),
            jax.ShapeDtypeStruct((EP, 128), jnp.float32),
        ],
    )(vs, ex16)


def _ln(h, g, b):
    mu = jnp.mean(h, axis=1, keepdims=True)
    var = jnp.mean((h - mu) ** 2, axis=1, keepdims=True)
    return (h - mu) / jnp.sqrt(var + 1e-5) * g + b


def _final_body(loc_ref, aggp_ref, den_ref, dis_ref, r_ref, bg_ref, wb_ref,
                g1_ref, b1_ref, g2_ref, b2_ref, wrel_ref, wroot_ref, lw_ref,
                gw_ref, out_ref):
    lp = loc_ref[...]
    local = jnp.concatenate([lp[0], lp[1]], axis=1)
    local = local * dis_ref[...] + bg_ref[...]
    ap = aggp_ref[...]
    agg = jnp.concatenate([ap[0, 0] + ap[1, 0], ap[0, 1] + ap[1, 1]], axis=1)
    den = den_ref[...][:, :H]
    dfac = jnp.concatenate(
        [jnp.broadcast_to(den[:, h:h + 1], (den.shape[0], Ch))
         for h in range(H)], axis=1)
    agg = agg / dfac
    rr = r_ref[...]
    wb = wb_ref[...]
    wa = wb[0:D] + wb[2 * D:3 * D]
    wr2 = wb[D:2 * D] - wb[2 * D:3 * D]
    z = (jnp.dot(agg, wa, preferred_element_type=jnp.float32)
         + jnp.dot(rr, wr2, preferred_element_type=jnp.float32))
    beta = jax.nn.sigmoid(z)
    glob = beta * rr + (1.0 - beta) * agg
    h = lw_ref[0, 0] * local + gw_ref[0, 0] * glob
    hln = _ln(h + h, g1_ref[...], b1_ref[...])
    f = jnp.maximum(
        jnp.dot(hln, wrel_ref[...], preferred_element_type=jnp.float32), 0.0)
    f = jnp.dot(f, wroot_ref[...], preferred_element_type=jnp.float32)
    out_ref[...] = _ln(f + hln, g2_ref[...], b2_ref[...])


def _final(loc, aggp, den, dis, rt, b_gcn, Wbeta, ln1_g, ln1_b, ln2_g, ln2_b,
           W_rel, W_root, lw, gw):
    fb = 400
    nb = N // fb
    return pl.pallas_call(
        _final_body,
        grid=(nb,),
        in_specs=[
            pl.BlockSpec((2, fb, 128), lambda i: (0, i, 0)),
            pl.BlockSpec((2, 2, fb, 128), lambda i: (0, 0, i, 0)),
            pl.BlockSpec((fb, 16), lambda i: (i, 0)),
            pl.BlockSpec((fb, 1), lambda i: (i, 0)),
            pl.BlockSpec((fb, D), lambda i: (i, 0)),
            pl.BlockSpec((1, D), lambda i: (0, 0)),
            pl.BlockSpec((3 * D, 1), lambda i: (0, 0)),
            pl.BlockSpec((1, D), lambda i: (0, 0)),
            pl.BlockSpec((1, D), lambda i: (0, 0)),
            pl.BlockSpec((1, D), lambda i: (0, 0)),
            pl.BlockSpec((1, D), lambda i: (0, 0)),
            pl.BlockSpec((D, 2 * D), lambda i: (0, 0)),
            pl.BlockSpec((2 * D, D), lambda i: (0, 0)),
            pl.BlockSpec((1, 1), lambda i: (0, 0)),
            pl.BlockSpec((1, 1), lambda i: (0, 0)),
        ],
        out_specs=pl.BlockSpec((fb, D), lambda i: (i, 0)),
        out_shape=jax.ShapeDtypeStruct((N, D), jnp.float32),
    )(loc, aggp, den, dis, rt, b_gcn.reshape(1, D), Wbeta,
      ln1_g.reshape(1, D), ln1_b.reshape(1, D), ln2_g.reshape(1, D),
      ln2_b.reshape(1, D), W_rel, W_root, lw.reshape(1, 1), gw.reshape(1, 1))


def kernel(x, edge_index, W_gcn, b_gcn, Wq, bq, Wk, bk, Wv, bv, Wskip, bskip,
           Wbeta, ln1_g, ln1_b, ln2_g, ln2_b, W_rel, W_root, lw, gw):
    src = edge_index[0]
    dst = edge_index[1]
    pad = jnp.full((EP - E,), N, jnp.int32)
    srcp = jnp.concatenate([src, pad])
    dstp = jnp.concatenate([dst, pad])
    xp = jnp.pad(x, ((0, NP - N), (0, 0)))
    Wcat = jnp.concatenate([W_gcn, Wq, Wk, Wv, Wskip], axis=1)
    Wcat = Wcat.reshape(D, 5, 2, 128).transpose(0, 2, 1, 3).reshape(D, 10 * 128)
    bcat = jnp.concatenate(
        [jnp.zeros_like(b_gcn), bq, bk, bv, bskip]).reshape(5, 2, 128)
    bcat = bcat.transpose(1, 0, 2).reshape(1, 10 * 128)

    xw_t, q_t, k_t, v_t, rt = _mm(xp, Wcat, bcat)
    degp = _sc_deg(dstp)
    dis, xs_t = _mid(degp, xw_t)
    qd, ks, vs = _sc_gather_qkv(dstp, srcp, q_t, k_t, v_t)
    loc = _sc_local(srcp, dstp, xs_t)
    lg = _logits(qd, ks)
    m = _gmax(lg)
    ex16 = _ex(lg, m)
    ssp = _sc_scatter16(dstp, ex16)
    den = _den(ssp)
    wv0, wv1 = _wv(vs, ex16)
    aggp = _sc_agg(dstp, wv0, wv1)
    return _final(loc, aggp, den, dis, rt, b_gcn, Wbeta, ln1_g, ln1_b, ln2_g,
                  ln2_b, W_rel, W_root, lw, gw)
